# baseline (speedup 1.0000x reference)
import functools

import jax
import jax.numpy as jnp
from jax import lax
from jax.experimental import pallas as pl
from jax.experimental.pallas import tpu as pltpu
from jax.experimental.pallas import tpu_sc as plsc

N, E, D, H, DE, T = 10000, 320000, 128, 128, 16, 4

NPAD = 10240
CHUNK = 64
NCORES, NSUB = 2, 16
NTILES = NCORES * NSUB
NC0, NC1 = 198, 118
EPAD = NSUB * (NC0 + NC1) * CHUNK
ROWS_PER_TILE = NPAD // NSUB
RB = 2000
EB = 3200
NEPB = 326400

def _dot(a, b):
    return jnp.dot(a, b, preferred_element_type=jnp.float32,
                   precision=lax.Precision.HIGHEST)


def _tc_pre_body(x_ref, we_ref, be_ref, wmx_ref, wmh_ref, wsx_ref, bu_ref,
                 xin_ref, hpx_ref, hp_ref, sxb_ref):
    xin = jnp.maximum(_dot(x_ref[...], we_ref[...]) + be_ref[...], 0.0)
    xin_ref[...] = xin
    hpx = _dot(xin, wmx_ref[...])
    hpx_ref[...] = hpx
    hp_ref[...] = hpx + _dot(xin, wmh_ref[...])
    sxb_ref[...] = _dot(xin, wsx_ref[...]) + bu_ref[...]


def _tc_pre(x, We, be, Wmx, Wmh, Wsx, bu):
    wspec = pl.BlockSpec((D, H), lambda i: (0, 0))
    bspec = pl.BlockSpec((1, H), lambda i: (0, 0))
    rspec = pl.BlockSpec((RB, D), lambda i: (i, 0))
    ospec = pl.BlockSpec((RB, H), lambda i: (i, 0))
    return pl.pallas_call(
        _tc_pre_body,
        grid=(N // RB,),
        in_specs=[rspec, wspec, bspec, wspec, wspec, wspec, bspec],
        out_specs=[ospec] * 4,
        out_shape=[jax.ShapeDtypeStruct((N, H), jnp.float32)] * 4,
    )(x, We, be, Wmx, Wmh, Wsx, bu)


def _tc_epb_body(ea_ref, we_ref, bm_ref, epb_ref):
    epb_ref[...] = _dot(ea_ref[...], we_ref[...]) + bm_ref[...]


def _tc_epb(ea, W_edge, bm):
    return pl.pallas_call(
        _tc_epb_body,
        grid=(E // EB,),
        in_specs=[pl.BlockSpec((EB, DE), lambda i: (i, 0)),
                  pl.BlockSpec((DE, H), lambda i: (0, 0)),
                  pl.BlockSpec((1, H), lambda i: (0, 0))],
        out_specs=pl.BlockSpec((EB, H), lambda i: (i, 0)),
        out_shape=jax.ShapeDtypeStruct((NEPB, H), jnp.float32),
    )(ea, W_edge, bm)


def _tc_step_body(aggp_ref, h_ref, hpx_ref, sxb_ref, wu_ref, wsh_ref, wmh_ref,
                  h2_ref, hp2_ref):
    agg = aggp_ref[0] + aggp_ref[1]
    h2 = jnp.maximum(
        _dot(agg, wu_ref[...]) + _dot(h_ref[...], wsh_ref[...]) + sxb_ref[...],
        0.0)
    h2_ref[...] = h2
    hp2_ref[...] = hpx_ref[...] + _dot(h2, wmh_ref[...])


def _tc_step(aggp, h, hpx, sxb, W_upd, Wsh, Wmh):
    wspec = pl.BlockSpec((H, H), lambda i: (0, 0))
    rspec = pl.BlockSpec((RB, H), lambda i: (i, 0))
    return pl.pallas_call(
        _tc_step_body,
        grid=(N // RB,),
        in_specs=[pl.BlockSpec((NCORES, RB, H), lambda i: (0, i, 0)),
                  rspec, rspec, rspec, wspec, wspec, wspec],
        out_specs=[rspec, rspec],
        out_shape=[jax.ShapeDtypeStruct((N, H), jnp.float32)] * 2,
    )(aggp, h, hpx, sxb, W_upd, Wsh, Wmh)


def _tc_last_body(aggp_ref, h_ref, xin_ref, sxb_ref, wu_ref, wsh_ref,
                  wdx_ref, wdh_ref, bd_ref, h2_ref, out_ref):
    agg = aggp_ref[0] + aggp_ref[1]
    h2 = jnp.maximum(
        _dot(agg, wu_ref[...]) + _dot(h_ref[...], wsh_ref[...]) + sxb_ref[...],
        0.0)
    h2_ref[...] = h2
    out_ref[...] = (_dot(xin_ref[...], wdx_ref[...]) +
                    _dot(h2, wdh_ref[...]) + bd_ref[...])


def _tc_last(aggp, h, xin, sxb, W_upd, Wsh, Wdx, Wdh, bd):
    wspec = pl.BlockSpec((H, H), lambda i: (0, 0))
    rspec = pl.BlockSpec((RB, H), lambda i: (i, 0))
    return pl.pallas_call(
        _tc_last_body,
        grid=(N // RB,),
        in_specs=[pl.BlockSpec((NCORES, RB, H), lambda i: (0, i, 0)),
                  rspec, rspec, rspec, wspec, wspec, wspec, wspec,
                  pl.BlockSpec((1, D), lambda i: (0, 0))],
        out_specs=[rspec, pl.BlockSpec((RB, D), lambda i: (i, 0))],
        out_shape=[jax.ShapeDtypeStruct((N, H), jnp.float32),
                   jax.ShapeDtypeStruct((N, D), jnp.float32)],
    )(aggp, h, xin, sxb, W_upd, Wsh, Wdx, Wdh, bd)


def _sc_agg_body(hp_hbm, sd_hbm, epb_hbm, out_hbm,
                 zbuf, sdv0, sdv1, gidx0, gidx1, sidx0, sidx1,
                 rows0, epbv0, rows1, epbv1, agg_sh,
                 semg0, seme0, semsc0, semg1, seme1, semsc1, semz):
    c = lax.axis_index("c")
    s = lax.axis_index("s")

    def _z(j, carry):
        for l in range(H // 16):
            zbuf[j, pl.ds(l * 16, 16)] = jnp.zeros((16,), jnp.float32)
        return carry
    lax.fori_loop(0, 32, _z, 0)

    def _zs(k, carry):
        pltpu.async_copy(zbuf, agg_sh.at[pl.ds(s * ROWS_PER_TILE + k * 32, 32)],
                         semz)
        return carry
    lax.fori_loop(0, ROWS_PER_TILE // 32, _zs, 0)

    def _zw(k, carry):
        pltpu.make_async_copy(zbuf, agg_sh.at[pl.ds(s * ROWS_PER_TILE, 32)],
                              semz).wait()
        return carry
    lax.fori_loop(0, ROWS_PER_TILE // 32, _zw, 0)
    plsc.subcore_barrier()

    cpt = jnp.where(c == 0, NC0, NC1)
    cb = c * (NSUB * NC0) + s * cpt
    base = cb * CHUNK

    def load_idx(g, sdv):
        pltpu.sync_copy(sd_hbm.at[cb + g], sdv)

    def start_gather(sdv, gidx, rows, semg):
        for l in range(CHUNK // 16):
            sl = pl.ds(l * 16, 16)
            gidx[sl] = sdv[sl]
        pltpu.async_copy(hp_hbm.at[gidx], rows, semg)

    def start_epb(eb, epbv, seme):
        pltpu.async_copy(epb_hbm.at[pl.ds(eb, CHUNK)], epbv, seme)

    def wait_gather(gidx, rows, semg):
        pltpu.make_async_copy(hp_hbm.at[gidx], rows, semg).wait()

    def wait_epb(epbv, seme):
        pltpu.make_async_copy(epb_hbm.at[pl.ds(0, CHUNK)], epbv, seme).wait()

    def compute(r, e):
        @plsc.parallel_loop(0, CHUNK)
        def _(i):
            for l in range(H // 16):
                sl = pl.ds(l * 16, 16)
                e[i, sl] = jnp.maximum(r[i, sl] + e[i, sl], 0.0)

    def start_scatter(sdv, sidx, m, semsc):
        for l in range(CHUNK // 16):
            sl = pl.ds(l * 16, 16)
            sidx[sl] = sdv[pl.ds(CHUNK + l * 16, 16)]
        pltpu.async_copy(m, agg_sh.at[sidx], semsc, add=True)

    def wait_scatter(sidx, m, semsc):
        pltpu.make_async_copy(m, agg_sh.at[sidx], semsc).wait()

    load_idx(0, sdv0)
    start_gather(sdv0, gidx0, rows0, semg0)
    start_epb(base, epbv0, seme0)

    def pair(p, carry):
        g1e = base + (2 * p + 1) * CHUNK
        g2e = base + (2 * p + 2) * CHUNK

        load_idx(2 * p + 1, sdv1)
        start_gather(sdv1, gidx1, rows1, semg1)

        @pl.when(p > 0)
        def _():
            wait_scatter(sidx1, epbv1, semsc1)
        start_epb(g1e, epbv1, seme1)

        wait_gather(gidx0, rows0, semg0)
        wait_epb(epbv0, seme0)
        compute(rows0, epbv0)
        start_scatter(sdv0, sidx0, epbv0, semsc0)

        @pl.when(p < cpt // 2 - 1)
        def _():
            load_idx(2 * p + 2, sdv0)
            start_gather(sdv0, gidx0, rows0, semg0)

        wait_gather(gidx1, rows1, semg1)
        wait_epb(epbv1, seme1)
        compute(rows1, epbv1)
        start_scatter(sdv1, sidx1, epbv1, semsc1)

        @pl.when(p < cpt // 2 - 1)
        def _():
            wait_scatter(sidx0, epbv0, semsc0)
            start_epb(g2e, epbv0, seme0)
        return carry

    lax.fori_loop(0, cpt // 2, pair, 0)
    wait_scatter(sidx0, epbv0, semsc0)
    wait_scatter(sidx1, epbv1, semsc1)

    plsc.subcore_barrier()
    pltpu.sync_copy(agg_sh.at[pl.ds(s * ROWS_PER_TILE, ROWS_PER_TILE)],
                    out_hbm.at[c, pl.ds(s * ROWS_PER_TILE, ROWS_PER_TILE)])


@functools.cache
def _make_sc_agg():
    return functools.partial(
        pl.kernel,
        out_type=jax.ShapeDtypeStruct((NCORES, NPAD, H), jnp.float32),
        mesh=plsc.VectorSubcoreMesh(core_axis_name="c", subcore_axis_name="s"),
        scratch_types=[
            pltpu.VMEM((32, H), jnp.float32),
            pltpu.VMEM((2 * CHUNK,), jnp.int32),
            pltpu.VMEM((2 * CHUNK,), jnp.int32),
            pltpu.VMEM((CHUNK,), jnp.int32),
            pltpu.VMEM((CHUNK,), jnp.int32),
            pltpu.VMEM((CHUNK,), jnp.int32),
            pltpu.VMEM((CHUNK,), jnp.int32),
            pltpu.VMEM((CHUNK, H), jnp.float32),
            pltpu.VMEM((CHUNK, H), jnp.float32),
            pltpu.VMEM((CHUNK, H), jnp.float32),
            pltpu.VMEM((CHUNK, H), jnp.float32),
            pltpu.VMEM_SHARED((NPAD, H), jnp.float32),
            pltpu.SemaphoreType.DMA,
            pltpu.SemaphoreType.DMA,
            pltpu.SemaphoreType.DMA,
            pltpu.SemaphoreType.DMA,
            pltpu.SemaphoreType.DMA,
            pltpu.SemaphoreType.DMA,
            pltpu.SemaphoreType.DMA,
        ],
    )(_sc_agg_body)


def kernel(x, edge_index, edge_attr, batch, W_enc, b_enc, W_msg, W_edge, b_msg,
           W_upd, W_self, b_upd, W_dec, b_dec):
    f32 = jnp.float32
    pad = EPAD - E
    src_p = jnp.concatenate([edge_index[0], jnp.zeros((pad,), jnp.int32)])
    dst_p = jnp.concatenate([edge_index[1], jnp.full((pad,), N, jnp.int32)])
    sd = jnp.concatenate([src_p.reshape(-1, CHUNK), dst_p.reshape(-1, CHUNK)],
                         axis=1)

    Wmx, Wmh = W_msg[:H], W_msg[H:2 * H] + W_msg[2 * H:]
    Wsx, Wsh = W_self[:H], W_self[H:2 * H] + W_self[2 * H:]
    Wdx, Wdh = W_dec[:H], W_dec[H:]
    be, bm = b_enc.reshape(1, H), b_msg.reshape(1, H)
    bu, bd = b_upd.reshape(1, H), b_dec.reshape(1, D)

    xin, hpx, hp, sxb = _tc_pre(x, W_enc, be, Wmx, Wmh, Wsx, bu)
    epb = _tc_epb(edge_attr, W_edge, bm)

    sc_agg = _make_sc_agg()
    h = xin
    for _ in range(T - 1):
        aggp = sc_agg(hp, sd, epb)
        h, hp = _tc_step(aggp, h, hpx, sxb, W_upd, Wsh, Wmh)

    aggp = sc_agg(hp, sd, epb)
    h, out = _tc_last(aggp, h, xin, sxb, W_upd, Wsh, Wdx, Wdh, bd)
    return (out, h)
